# baseline (speedup 1.0000x reference)
import functools

import jax
import jax.numpy as jnp
from jax import lax
from jax.experimental import pallas as pl
from jax.experimental.pallas import tpu as pltpu
from jax.experimental.pallas import tpu_sc as plsc


def _build(B, S, D, V):
    NW = 32
    CH = 512
    TPC = CH // 128
    CPS = B // CH
    SB = B // 128
    n_chunks = S * CPS
    chunks_per_w = n_chunks // NW
    user_per_w = B // NW
    JT = D // 8

    mesh = plsc.VectorSubcoreMesh(core_axis_name="c", subcore_axis_name="s")

    @functools.partial(
        pl.kernel,
        mesh=mesh,
        out_type=[
            jax.ShapeDtypeStruct((JT, SB, 8, 128), jnp.float32),
            jax.ShapeDtypeStruct((S, JT, SB, 8, 128), jnp.float32),
        ],
        scratch_types=[
            pltpu.VMEM((user_per_w,), jnp.int32),
            pltpu.VMEM((user_per_w, D), jnp.float32),
            pltpu.VMEM((CH,), jnp.int32),
            pltpu.VMEM((CH,), jnp.int32),
            pltpu.VMEM((CH, D), jnp.float32),
            pltpu.VMEM((CH, D), jnp.float32),
            pltpu.VMEM((JT, TPC, 8, 128), jnp.float32),
            pltpu.VMEM((JT, TPC, 8, 128), jnp.float32),
            pltpu.SemaphoreType.DMA,
            pltpu.SemaphoreType.DMA,
            pltpu.SemaphoreType.DMA,
            pltpu.SemaphoreType.DMA,
            pltpu.SemaphoreType.DMA,
            pltpu.SemaphoreType.DMA,
            pltpu.SemaphoreType.DMA,
        ],
        compiler_params=pltpu.CompilerParams(use_tc_tiling_on_sc=False),
    )
    def k(uids, iids_t, utab, itab, uout, iout5,
          uidx_v, urows_v, idx0, idx1, rows0, rows1, out0, out1,
          usem, sl0, sl1, sg0, sg1, sw0, sw1):
        wid = lax.axis_index("s") * 2 + lax.axis_index("c")

        iota = lax.iota(jnp.int32, 16)
        masks = {d: (iota & d) == 0 for d in (8, 4, 2, 1)}
        perms = {d: iota ^ d for d in (8, 4, 2, 1)}

        def transpose_chunk(rows_v, out_v):
            for btl in range(TPC):
                def tbody(t, carry):
                    bb = btl * 128 + t * 16
                    for jg in range(D // 16):
                        a = [rows_v[bb + r, pl.ds(jg * 16, 16)]
                             for r in range(16)]
                        for d in (8, 4, 2, 1):
                            for i in range(16):
                                if i & d:
                                    continue
                                A, B = a[i], a[i + d]
                                Dx = jnp.where(masks[d], B, A)[perms[d]]
                                a[i] = jnp.where(masks[d], A, Dx)
                                a[i + d] = jnp.where(masks[d], Dx, B)
                        for jj in range(16):
                            jgl = jg * 16 + jj
                            out_v[jgl // 8, btl, jgl % 8,
                                  pl.ds(t * 16, 16)] = a[jj]
                    return carry
                lax.fori_loop(0, 8, tbody, 0)

        ubase = wid * user_per_w
        pltpu.sync_copy(uids.at[pl.ds(ubase, user_per_w)], uidx_v)
        pltpu.async_copy(utab.at[uidx_v], urows_v, usem).wait()
        transpose_chunk(urows_v, out0)
        for jt in range(JT):
            pltpu.make_async_copy(
                out0.at[jt], uout.at[jt, pl.ds(wid * TPC, TPC)], usem).start()
        for jt in range(JT):
            pltpu.make_async_copy(
                out0.at[jt], uout.at[jt, pl.ds(wid * TPC, TPC)], usem).wait()

        c0 = wid * chunks_per_w
        last = n_chunks - 1

        def parts(c):
            cc = jnp.minimum(c, last)
            return cc // CPS, (cc % CPS) * TPC

        def load(c, idx_v, sem):
            s, btl0 = parts(c)
            return pltpu.make_async_copy(
                iids_t.at[s, pl.ds(btl0 * 128, CH)], idx_v, sem)

        def gath(idx_v, rows_v, sem):
            return pltpu.make_async_copy(itab.at[idx_v], rows_v, sem)

        def wr(c, out_v, sem, wait):
            s, btl0 = parts(c)
            for jt in range(JT):
                cp = pltpu.make_async_copy(
                    out_v.at[jt], iout5.at[s, jt, pl.ds(btl0, TPC)], sem)
                if wait:
                    cp.wait()
                else:
                    cp.start()

        load(c0, idx0, sl0).start()
        load(c0, idx0, sl0).wait()
        gath(idx0, rows0, sg0).start()
        load(c0 + 1, idx1, sl1).start()
        load(c0 + 1, idx1, sl1).wait()
        gath(idx1, rows1, sg1).start()
        wr(c0, out0, sw0, False)
        wr(c0 + 1, out1, sw1, False)

        def body(j, carry):
            c = c0 + 2 * j
            gath(idx0, rows0, sg0).wait()
            load(c + 2, idx0, sl0).start()
            wr(c, out0, sw0, True)
            transpose_chunk(rows0, out0)
            wr(c, out0, sw0, False)
            load(c + 2, idx0, sl0).wait()
            gath(idx0, rows0, sg0).start()
            gath(idx1, rows1, sg1).wait()
            load(c + 3, idx1, sl1).start()
            wr(c + 1, out1, sw1, True)
            transpose_chunk(rows1, out1)
            wr(c + 1, out1, sw1, False)
            load(c + 3, idx1, sl1).wait()
            gath(idx1, rows1, sg1).start()
            return carry

        lax.fori_loop(0, chunks_per_w // 2, body, 0)

        gath(idx0, rows0, sg0).wait()
        gath(idx1, rows1, sg1).wait()
        wr(c0 + chunks_per_w - 2, out0, sw0, True)
        wr(c0 + chunks_per_w - 1, out1, sw1, True)

    return k


def kernel(user_ids, item_ids, user_table, item_table):
    B, S = item_ids.shape
    V, D = user_table.shape
    k = _build(B, S, D, V)
    uout5, iout5 = k(user_ids, item_ids.T, user_table, item_table)
    user_emb = uout5.transpose(1, 3, 0, 2).reshape(B, D)
    item_emb = iout5.transpose(2, 4, 0, 1, 3).reshape(B, S, D)
    return user_emb, item_emb


# speedup vs baseline: 1.0043x; 1.0043x over previous
import functools

import jax
import jax.numpy as jnp
from jax import lax
from jax.experimental import pallas as pl
from jax.experimental.pallas import tpu as pltpu
from jax.experimental.pallas import tpu_sc as plsc

_NW = 32


def _mesh():
    return plsc.VectorSubcoreMesh(core_axis_name="c", subcore_axis_name="s")


def _tbody_factory(rows_v, out_v, TPC, D, masks, perms, iota):
    del iota

    def run():
        for btl in range(TPC):
            def tbody(t, carry):
                bb = btl * 128 + t * 16
                for jg in range(D // 16):
                    a = [rows_v[bb + r, pl.ds(jg * 16, 16)]
                         for r in range(16)]
                    for d in (8, 4, 2, 1):
                        for i in range(16):
                            if i & d:
                                continue
                            A, B = a[i], a[i + d]
                            Dx = jnp.where(masks[d], B, A)[perms[d]]
                            a[i] = jnp.where(masks[d], A, Dx)
                            a[i + d] = jnp.where(masks[d], Dx, B)
                    for jj in range(16):
                        jgl = jg * 16 + jj
                        out_v[jgl // 8, btl, jgl % 8,
                              pl.ds(t * 16, 16)] = a[jj]
                return carry
            lax.fori_loop(0, 8, tbody, 0)

    return run


def _build_user(B, D, V):
    per_w = B // _NW
    TPC = per_w // 128
    JT = D // 8

    @functools.partial(
        pl.kernel,
        mesh=_mesh(),
        out_type=[jax.ShapeDtypeStruct((JT, B // 128, 8, 128), jnp.float32)],
        scratch_types=[
            pltpu.VMEM((per_w,), jnp.int32),
            pltpu.VMEM((per_w, D), jnp.float32),
            pltpu.VMEM((JT, TPC, 8, 128), jnp.float32),
            pltpu.SemaphoreType.DMA,
        ],
        compiler_params=pltpu.CompilerParams(use_tc_tiling_on_sc=False),
    )
    def k(uids, utab, uout, uidx_v, urows_v, out_v, sem):
        wid = lax.axis_index("s") * 2 + lax.axis_index("c")
        iota = lax.iota(jnp.int32, 16)
        masks = {d: (iota & d) == 0 for d in (8, 4, 2, 1)}
        perms = {d: iota ^ d for d in (8, 4, 2, 1)}
        ubase = wid * per_w
        pltpu.sync_copy(uids.at[pl.ds(ubase, per_w)], uidx_v)
        pltpu.async_copy(utab.at[uidx_v], urows_v, sem).wait()
        _tbody_factory(urows_v, out_v, TPC, D, masks, perms, iota)()
        for jt in range(JT):
            pltpu.make_async_copy(
                out_v.at[jt], uout.at[jt, pl.ds(wid * TPC, TPC)], sem).start()
        for jt in range(JT):
            pltpu.make_async_copy(
                out_v.at[jt], uout.at[jt, pl.ds(wid * TPC, TPC)], sem).wait()

    return k


def _build_item(B, S, D, V):
    CH = 512
    TPC = CH // 128
    CPS = B // CH
    SB = B // 128
    n_chunks = S * CPS
    chunks_per_w = n_chunks // _NW
    JT = D // 8

    @functools.partial(
        pl.kernel,
        mesh=_mesh(),
        out_type=[jax.ShapeDtypeStruct((S, JT, SB, 8, 128), jnp.float32)],
        scratch_types=[
            pltpu.VMEM((CH,), jnp.int32),
            pltpu.VMEM((CH,), jnp.int32),
            pltpu.VMEM((CH, D), jnp.float32),
            pltpu.VMEM((CH, D), jnp.float32),
            pltpu.VMEM((JT, TPC, 8, 128), jnp.float32),
            pltpu.VMEM((JT, TPC, 8, 128), jnp.float32),
            pltpu.SemaphoreType.DMA,
            pltpu.SemaphoreType.DMA,
            pltpu.SemaphoreType.DMA,
            pltpu.SemaphoreType.DMA,
            pltpu.SemaphoreType.DMA,
            pltpu.SemaphoreType.DMA,
        ],
        compiler_params=pltpu.CompilerParams(use_tc_tiling_on_sc=False),
    )
    def k(iids_t, itab, iout5,
          idx0, idx1, rows0, rows1, out0, out1,
          sl0, sl1, sg0, sg1, sw0, sw1):
        wid = lax.axis_index("s") * 2 + lax.axis_index("c")
        iota = lax.iota(jnp.int32, 16)
        masks = {d: (iota & d) == 0 for d in (8, 4, 2, 1)}
        perms = {d: iota ^ d for d in (8, 4, 2, 1)}

        c0 = wid * chunks_per_w
        last = n_chunks - 1

        def parts(c):
            cc = jnp.minimum(c, last)
            return cc // CPS, (cc % CPS) * TPC

        def load(c, idx_v, sem):
            s, btl0 = parts(c)
            return pltpu.make_async_copy(
                iids_t.at[s, pl.ds(btl0 * 128, CH)], idx_v, sem)

        def gath(idx_v, rows_v, sem):
            return pltpu.make_async_copy(itab.at[idx_v], rows_v, sem)

        def wr(c, out_v, sem, wait):
            s, btl0 = parts(c)
            for jt in range(JT):
                cp = pltpu.make_async_copy(
                    out_v.at[jt], iout5.at[s, jt, pl.ds(btl0, TPC)], sem)
                if wait:
                    cp.wait()
                else:
                    cp.start()

        load(c0, idx0, sl0).start()
        load(c0, idx0, sl0).wait()
        gath(idx0, rows0, sg0).start()
        load(c0 + 1, idx1, sl1).start()
        load(c0 + 1, idx1, sl1).wait()
        gath(idx1, rows1, sg1).start()
        wr(c0, out0, sw0, False)
        wr(c0 + 1, out1, sw1, False)

        def body(j, carry):
            c = c0 + 2 * j
            gath(idx0, rows0, sg0).wait()
            load(c + 2, idx0, sl0).start()
            wr(c, out0, sw0, True)
            _tbody_factory(rows0, out0, TPC, D, masks, perms, iota)()
            wr(c, out0, sw0, False)
            load(c + 2, idx0, sl0).wait()
            gath(idx0, rows0, sg0).start()
            gath(idx1, rows1, sg1).wait()
            load(c + 3, idx1, sl1).start()
            wr(c + 1, out1, sw1, True)
            _tbody_factory(rows1, out1, TPC, D, masks, perms, iota)()
            wr(c + 1, out1, sw1, False)
            load(c + 3, idx1, sl1).wait()
            gath(idx1, rows1, sg1).start()
            return carry

        lax.fori_loop(0, chunks_per_w // 2, body, 0)

        gath(idx0, rows0, sg0).wait()
        gath(idx1, rows1, sg1).wait()
        wr(c0 + chunks_per_w - 2, out0, sw0, True)
        wr(c0 + chunks_per_w - 1, out1, sw1, True)

    return k


def kernel(user_ids, item_ids, user_table, item_table):
    B, S = item_ids.shape
    V, D = user_table.shape
    iout5 = _build_item(B, S, D, V)(item_ids.T, item_table)
    uout5 = _build_user(B, D, V)(user_ids, user_table)
    if isinstance(iout5, (tuple, list)):
        iout5 = iout5[0]
    if isinstance(uout5, (tuple, list)):
        uout5 = uout5[0]
    user_emb = uout5.transpose(1, 3, 0, 2).reshape(B, D)
    item_emb = iout5.transpose(2, 4, 0, 1, 3).reshape(B, S, D)
    return user_emb, item_emb
